# Initial kernel scaffold; baseline (speedup 1.0000x reference)
import jax
import jax.numpy as jnp
from jax.experimental import pallas as pl


def kernel(x, edge_index, batch_ids, W1, W1s, b1, g1, be1, W2, W2s, b2, g2, be2, W3, W3s, b3, g3, be3, Wl, bl):
    raise NotImplementedError("write your pallas kernel here")


# speedup vs baseline: 18.7329x; 18.7329x over previous
import jax
import jax.numpy as jnp
from jax import lax
from jax.experimental import pallas as pl
from jax.experimental.pallas import tpu as pltpu
from jax.experimental.pallas import tpu_sc as plsc

N0, N1, N2, N3, N4, N5, N6 = 10000, 5000, 2500, 1250, 625, 313, 157
CIN = 128
NE = 320000
NW = 32
EPT = 10240
NCHUNK = EPT // 128
NEP = EPT * NW
PAD_PER_TILE = EPT - NE // NW

S1_ROWS = 5120
S1_STRIPE = S1_ROWS // 16
C3_FLAT = 204800
C3_STRIPE = C3_FLAT // 16
S2_ROWS = 1280
H2_ROWS = 2560

_i32 = jnp.int32

_MESH = plsc.VectorSubcoreMesh(core_axis_name="c", subcore_axis_name="s")


def _sc_conv1_body(x_hbm, sd_hbm, z2d_hbm,
                   s1_out,
                   srcv, d1b, rows0, rows1, zb,
                   s1acc, sem0, sem1):
    cid = lax.axis_index("c")
    sid = lax.axis_index("s")
    wid = sid * _i32(2) + cid
    base = wid * _i32(EPT)

    pltpu.sync_copy(sd_hbm.at[pl.ds(base, EPT)], srcv)
    pltpu.sync_copy(z2d_hbm, zb)

    r0 = sid * _i32(S1_STRIPE)
    pltpu.sync_copy(zb, s1acc.at[pl.ds(r0, 128)])
    pltpu.sync_copy(zb, s1acc.at[pl.ds(r0 + _i32(128), 128)])
    pltpu.sync_copy(zb.at[pl.ds(0, S1_STRIPE - 256)],
                    s1acc.at[pl.ds(r0 + _i32(256), S1_STRIPE - 256)])

    def idx_body(j, _):
        off = pl.multiple_of(j * _i32(128), 128)
        for k in range(8):
            pck = srcv[pl.ds(off + _i32(k * 16), 16)]
            d = lax.shift_right_logical(pck, _i32(14))
            d1b[j, pl.ds(k * 16, 16)] = lax.shift_right_logical(d, _i32(1))
            srcv[pl.ds(off + _i32(k * 16), 16)] = pck & _i32(0x3FFF)
        return _i32(0)

    lax.fori_loop(_i32(0), _i32(NCHUNK), idx_body, _i32(0))
    plsc.subcore_barrier()

    def gather(j, buf, sem):
        off = pl.multiple_of(j * _i32(128), 128)
        pltpu.async_copy(x_hbm.at[srcv.at[pl.ds(off, 128)]], buf, sem)

    def wait(buf, sem):
        pltpu.make_async_copy(x_hbm.at[pl.ds(0, 128)], buf, sem).wait()

    def scatter(j, buf):
        pltpu.sync_copy(buf, s1acc.at[d1b.at[j]], add=True)

    gather(_i32(0), rows0, sem0)

    def main_body(i, _):
        j0 = i * _i32(2)
        gather(j0 + _i32(1), rows1, sem1)
        wait(rows0, sem0)
        scatter(j0, rows0)

        @pl.when(i < _i32(NCHUNK // 2 - 1))
        def _():
            gather(j0 + _i32(2), rows0, sem0)

        wait(rows1, sem1)
        scatter(j0 + _i32(1), rows1)
        return _i32(0)

    lax.fori_loop(_i32(0), _i32(NCHUNK // 2), main_body, _i32(0))
    plsc.subcore_barrier()

    pltpu.sync_copy(s1acc.at[pl.ds(r0, S1_STRIPE)],
                    s1_out.at[cid, pl.ds(r0, S1_STRIPE)])


@jax.jit
def _sc_conv1(x, sd_p):
    z2d = jnp.zeros((128, CIN), jnp.float32)
    return pl.kernel(
        _sc_conv1_body,
        out_type=jax.ShapeDtypeStruct((2, S1_ROWS, CIN), jnp.float32),
        mesh=_MESH,
        scratch_types=[
            pltpu.VMEM((EPT,), jnp.int32),
            pltpu.VMEM((NCHUNK, 128), jnp.int32),
            pltpu.VMEM((128, CIN), jnp.float32),
            pltpu.VMEM((128, CIN), jnp.float32),
            pltpu.VMEM((128, CIN), jnp.float32),
            pltpu.VMEM_SHARED((S1_ROWS, CIN), jnp.float32),
            pltpu.SemaphoreType.DMA,
            pltpu.SemaphoreType.DMA,
        ],
    )(x, sd_p, z2d)


def _sc_conv2_body(h2_hbm, sd_hbm, z2d_hbm, z1d_hbm,
                   s2_out, c3_out,
                   srcv, d2b, c3b, rows0, rows1, onesv, zb, zb1,
                   s2acc, c3acc, sem0, sem1):
    cid = lax.axis_index("c")
    sid = lax.axis_index("s")
    wid = sid * _i32(2) + cid
    base = wid * _i32(EPT)

    pltpu.sync_copy(sd_hbm.at[pl.ds(base, EPT)], srcv)
    pltpu.sync_copy(z2d_hbm, zb)
    pltpu.sync_copy(z1d_hbm, zb1)
    for k in range(8):
        onesv[pl.ds(k * 16, 16)] = jnp.ones((16,), jnp.float32)

    r0 = sid * _i32(S2_ROWS // 16)
    pltpu.sync_copy(zb.at[pl.ds(0, S2_ROWS // 16)],
                    s2acc.at[pl.ds(r0, S2_ROWS // 16)])
    f0 = sid * _i32(C3_STRIPE)
    pltpu.sync_copy(zb1, c3acc.at[pl.ds(f0, C3_STRIPE)])

    def idx_body(j, _):
        off = pl.multiple_of(j * _i32(128), 128)
        for k in range(8):
            pck = srcv[pl.ds(off + _i32(k * 16), 16)]
            sv = pck & _i32(0x3FFF)
            d = lax.shift_right_logical(pck, _i32(14))
            srcv[pl.ds(off + _i32(k * 16), 16)] = lax.shift_right_logical(
                sv, _i32(2))
            d2b[j, pl.ds(k * 16, 16)] = lax.shift_right_logical(d, _i32(3))
            s3 = lax.shift_right_logical(sv, _i32(4))
            c3 = lax.shift_right_logical(d, _i32(5)) * _i32(625) + s3
            c3b[j, pl.ds(k * 16, 16)] = jnp.where(
                d < _i32(N0), c3, _i32(196000) + s3)
        return _i32(0)

    lax.fori_loop(_i32(0), _i32(NCHUNK), idx_body, _i32(0))
    plsc.subcore_barrier()

    def gather(j, buf, sem):
        off = pl.multiple_of(j * _i32(128), 128)
        pltpu.async_copy(h2_hbm.at[srcv.at[pl.ds(off, 128)]], buf, sem)

    def wait(buf, sem):
        pltpu.make_async_copy(h2_hbm.at[pl.ds(0, 128)], buf, sem).wait()

    def scatter(j, buf):
        pltpu.sync_copy(buf, s2acc.at[d2b.at[j]], add=True)
        pltpu.sync_copy(onesv, c3acc.at[c3b.at[j]], add=True)

    gather(_i32(0), rows0, sem0)

    def main_body(i, _):
        j0 = i * _i32(2)
        gather(j0 + _i32(1), rows1, sem1)
        wait(rows0, sem0)
        scatter(j0, rows0)

        @pl.when(i < _i32(NCHUNK // 2 - 1))
        def _():
            gather(j0 + _i32(2), rows0, sem0)

        wait(rows1, sem1)
        scatter(j0 + _i32(1), rows1)
        return _i32(0)

    lax.fori_loop(_i32(0), _i32(NCHUNK // 2), main_body, _i32(0))
    plsc.subcore_barrier()

    pltpu.sync_copy(s2acc.at[pl.ds(r0, S2_ROWS // 16)],
                    s2_out.at[cid, pl.ds(r0, S2_ROWS // 16)])
    pltpu.sync_copy(c3acc.at[pl.ds(f0, C3_STRIPE)],
                    c3_out.at[cid, pl.ds(f0, C3_STRIPE)])


@jax.jit
def _sc_conv2(h2p, sd_p):
    z2d = jnp.zeros((128, CIN), jnp.float32)
    z1d = jnp.zeros((C3_STRIPE,), jnp.float32)
    return pl.kernel(
        _sc_conv2_body,
        out_type=(
            jax.ShapeDtypeStruct((2, S2_ROWS, CIN), jnp.float32),
            jax.ShapeDtypeStruct((2, C3_FLAT), jnp.float32),
        ),
        mesh=_MESH,
        scratch_types=[
            pltpu.VMEM((EPT,), jnp.int32),
            pltpu.VMEM((NCHUNK, 128), jnp.int32),
            pltpu.VMEM((NCHUNK, 128), jnp.int32),
            pltpu.VMEM((128, CIN), jnp.float32),
            pltpu.VMEM((128, CIN), jnp.float32),
            pltpu.VMEM((128,), jnp.float32),
            pltpu.VMEM((128, CIN), jnp.float32),
            pltpu.VMEM((C3_STRIPE,), jnp.float32),
            pltpu.VMEM_SHARED((S2_ROWS, CIN), jnp.float32),
            pltpu.VMEM_SHARED((C3_FLAT,), jnp.float32),
            pltpu.SemaphoreType.DMA,
            pltpu.SemaphoreType.DMA,
        ],
    )(h2p, sd_p, z2d, z1d)


def _bn_relu(h, g, b):
    mu = jnp.mean(h, axis=0, keepdims=True)
    var = jnp.mean((h - mu) ** 2, axis=0, keepdims=True)
    return jnp.maximum((h - mu) * lax.rsqrt(var + 1e-5) * g + b, 0.0)


def _tc_b_body(s1_ref, xp_ref, w1_ref, w1s_ref, b1_ref, g1_ref, be1_ref,
               out_ref):
    s1 = s1_ref[0, :N1, :] + s1_ref[1, :N1, :]
    xp = xp_ref[...]
    px = xp[:, :CIN] + xp[:, CIN:]
    h = (jnp.dot(s1, w1_ref[...], preferred_element_type=jnp.float32,
             precision=lax.Precision.HIGHEST)
         + jnp.dot(px, w1s_ref[...], preferred_element_type=jnp.float32,
             precision=lax.Precision.HIGHEST)
         + b1_ref[...])
    out_ref[...] = _bn_relu(h, g1_ref[...], be1_ref[...])


@jax.jit
def _tc_b(s1p, xp, w1, w1s, b1, g1, be1):
    return pl.pallas_call(
        _tc_b_body,
        out_shape=jax.ShapeDtypeStruct((N1, 64), jnp.float32),
    )(s1p, xp, w1, w1s, b1, g1, be1)


def _tc_d1_body(s2_ref, h1q_ref, w2_ref, w2s_ref, b2_ref, g2_ref, be2_ref,
                out_ref):
    f32 = jnp.float32
    s2w = s2_ref[0, :N3, :] + s2_ref[1, :N3, :]
    s2 = (s2w[:, :64] + s2w[:, 64:]) * 0.5
    h1q = h1q_ref[...]
    p2 = (h1q[:, :64] + h1q[:, 64:128] + h1q[:, 128:192]
          + h1q[:, 192:]) * 0.5
    h = (jnp.dot(s2, w2_ref[...], preferred_element_type=f32,
             precision=lax.Precision.HIGHEST)
         + jnp.dot(p2, w2s_ref[...], preferred_element_type=f32,
             precision=lax.Precision.HIGHEST)
         + b2_ref[...])
    out_ref[...] = _bn_relu(h, g2_ref[...], be2_ref[...])


@jax.jit
def _tc_d1(s2p, h1q, w2, w2s, b2, g2, be2):
    return pl.pallas_call(
        _tc_d1_body,
        out_shape=jax.ShapeDtypeStruct((N3, 128), jnp.float32),
    )(s2p, h1q, w2, w2s, b2, g2, be2)


def _tc_d2_body(h3r_ref, h3q_ref, c3_ref, w3_ref, w3s_ref, b3_ref, g3_ref,
                be3_ref, out_ref):
    f32 = jnp.float32
    h3r = h3r_ref[...]
    h4 = (h3r[:, :128] + h3r[:, 128:]) * 0.5
    c3 = c3_ref[0] + c3_ref[1]
    m = jnp.dot(c3, h4, preferred_element_type=f32,
             precision=lax.Precision.HIGHEST)
    h3q = h3q_ref[...]
    p4 = (h3q[:, :128] + h3q[:, 128:256] + h3q[:, 256:384]
          + h3q[:, 384:]) * 0.5
    h = (jnp.dot(m, w3_ref[...], preferred_element_type=f32,
             precision=lax.Precision.HIGHEST)
         + jnp.dot(p4, w3s_ref[...], preferred_element_type=f32,
             precision=lax.Precision.HIGHEST)
         + b3_ref[...])
    out_ref[...] = _bn_relu(h, g3_ref[...], be3_ref[...])


@jax.jit
def _tc_d2(h3r, h3q, c3p, w3, w3s, b3, g3, be3):
    return pl.pallas_call(
        _tc_d2_body,
        out_shape=jax.ShapeDtypeStruct((N5, 256), jnp.float32),
    )(h3r, h3q, c3p, w3, w3s, b3, g3, be3)


def _tc_d3_body(gq_ref, b6_ref, wl_ref, bl_ref, out_ref):
    f32 = jnp.float32
    gq = gq_ref[...]
    s6 = gq[:, :256] + gq[:, 256:]
    ridx = lax.broadcasted_iota(jnp.int32, (N6, 1), 0)
    h6 = s6 * jnp.where(ridx < _i32(N6 - 1), f32(0.5), f32(1.0))
    b6 = b6_ref[...]
    parts = []
    for b in range(8):
        mb = jnp.where(b6 == _i32(b), h6, -jnp.inf)
        parts.append(jnp.max(mb, axis=0, keepdims=True))
    pooled = jnp.concatenate(parts, axis=0)
    pooled = jnp.where(jnp.isfinite(pooled), pooled, f32(0.0))
    out_ref[...] = (jnp.dot(pooled, wl_ref[...], preferred_element_type=f32,
             precision=lax.Precision.HIGHEST)
                    + bl_ref[...])


@jax.jit
def _tc_d3(gq, b6, wl, bl):
    return pl.pallas_call(
        _tc_d3_body,
        out_shape=jax.ShapeDtypeStruct((8, 40), jnp.float32),
    )(gq, b6, wl, bl)


def kernel(x, edge_index, batch_ids, W1, W1s, b1, g1, be1, W2, W2s, b2, g2,
           be2, W3, W3s, b3, g3, be3, Wl, bl):
    x = x.astype(jnp.float32)
    src = edge_index[0].astype(jnp.int32).reshape(NW, NE // NW)
    dst = edge_index[1].astype(jnp.int32).reshape(NW, NE // NW)

    k = jnp.arange(PAD_PER_TILE, dtype=jnp.int32)
    pad_src = jnp.broadcast_to(k % N0, (NW, PAD_PER_TILE))
    pad_dst = jnp.broadcast_to(N0 + (k % 16), (NW, PAD_PER_TILE))
    src_p = jnp.concatenate([src, pad_src], axis=1).reshape(NEP)
    dst_p = jnp.concatenate([dst, pad_dst], axis=1).reshape(NEP)
    sd_p = src_p | (dst_p << 14)

    s1p = _sc_conv1(x, sd_p)

    xp = x.reshape(N1, 2 * CIN)
    h1 = _tc_b(s1p, xp, W1, W1s, b1.reshape(1, 64), g1.reshape(1, 64),
               be1.reshape(1, 64))

    h1r = h1.reshape(N2, 128)
    h1q = h1.reshape(N3, 256)

    s2p, c3p = _sc_conv2(h1r, sd_p)
    c3m = c3p[:, :N5 * N4].reshape(2, N5, N4)

    h3 = _tc_d1(s2p, h1q, W2, W2s, b2.reshape(1, 128), g2.reshape(1, 128),
                be2.reshape(1, 128))
    h3r = h3.reshape(N4, 256)
    h3q = jnp.concatenate(
        [h3, jnp.zeros((2, 128), jnp.float32)], axis=0).reshape(N5, 512)

    gg = _tc_d2(h3r, h3q, c3m, W3, W3s, b3.reshape(1, 256),
                g3.reshape(1, 256), be3.reshape(1, 256))
    gq = jnp.concatenate(
        [gg, jnp.zeros((1, 256), jnp.float32)], axis=0).reshape(N6, 512)

    b6 = batch_ids[::64][:N6].astype(jnp.int32).reshape(N6, 1)
    return _tc_d3(gq, b6, Wl, bl.reshape(1, 40))


# speedup vs baseline: 20.4813x; 1.0933x over previous
import jax
import jax.numpy as jnp
from jax import lax
from jax.experimental import pallas as pl
from jax.experimental.pallas import tpu as pltpu
from jax.experimental.pallas import tpu_sc as plsc

N0, N1, N2, N3, N4, N5, N6 = 10000, 5000, 2500, 1250, 625, 313, 157
CIN = 128
NE = 320000
NW = 32
EPT = 10240
NCHUNK = EPT // 128
NEP = EPT * NW
PAD_PER_TILE = EPT - NE // NW

S1_ROWS = 5120
S1_STRIPE = S1_ROWS // 16
C3_FLAT = 204800
C3_STRIPE = C3_FLAT // 16
S2_ROWS = 1280
H2_ROWS = 2560

_i32 = jnp.int32

_MESH = plsc.VectorSubcoreMesh(core_axis_name="c", subcore_axis_name="s")


def _sc_conv1_body(x_hbm, sd_hbm, z2d_hbm,
                   s1_out,
                   srcv, d1b, rows0, rows1, zb,
                   s1acc, sem0, sem1):
    cid = lax.axis_index("c")
    sid = lax.axis_index("s")
    wid = sid * _i32(2) + cid
    base = wid * _i32(EPT)

    pltpu.sync_copy(sd_hbm.at[pl.ds(base, EPT)], srcv)
    pltpu.sync_copy(z2d_hbm, zb)

    r0 = sid * _i32(S1_STRIPE)
    pltpu.sync_copy(zb, s1acc.at[pl.ds(r0, 128)])
    pltpu.sync_copy(zb, s1acc.at[pl.ds(r0 + _i32(128), 128)])
    pltpu.sync_copy(zb.at[pl.ds(0, S1_STRIPE - 256)],
                    s1acc.at[pl.ds(r0 + _i32(256), S1_STRIPE - 256)])

    def idx_body(j, _):
        off = pl.multiple_of(j * _i32(128), 128)
        for k in range(8):
            pck = srcv[pl.ds(off + _i32(k * 16), 16)]
            d = lax.shift_right_logical(pck, _i32(14))
            d1b[j, pl.ds(k * 16, 16)] = lax.shift_right_logical(d, _i32(1))
            srcv[pl.ds(off + _i32(k * 16), 16)] = pck & _i32(0x3FFF)
        return _i32(0)

    lax.fori_loop(_i32(0), _i32(NCHUNK), idx_body, _i32(0))
    plsc.subcore_barrier()

    def gather(j, buf, sem):
        off = pl.multiple_of(j * _i32(128), 128)
        pltpu.async_copy(x_hbm.at[srcv.at[pl.ds(off, 128)]], buf, sem)

    def wait(buf, sem):
        pltpu.make_async_copy(x_hbm.at[pl.ds(0, 128)], buf, sem).wait()

    def scatter(j, buf):
        pltpu.sync_copy(buf, s1acc.at[d1b.at[j]], add=True)

    gather(_i32(0), rows0, sem0)

    def main_body(i, _):
        j0 = i * _i32(2)
        gather(j0 + _i32(1), rows1, sem1)
        wait(rows0, sem0)
        scatter(j0, rows0)

        @pl.when(i < _i32(NCHUNK // 2 - 1))
        def _():
            gather(j0 + _i32(2), rows0, sem0)

        wait(rows1, sem1)
        scatter(j0 + _i32(1), rows1)
        return _i32(0)

    lax.fori_loop(_i32(0), _i32(NCHUNK // 2), main_body, _i32(0))
    plsc.subcore_barrier()

    pltpu.sync_copy(s1acc.at[pl.ds(r0, S1_STRIPE)],
                    s1_out.at[cid, pl.ds(r0, S1_STRIPE)])


@jax.jit
def _sc_conv1(x, sd_p):
    z2d = jnp.zeros((128, CIN), jnp.float32)
    return pl.kernel(
        _sc_conv1_body,
        out_type=jax.ShapeDtypeStruct((2, S1_ROWS, CIN), jnp.float32),
        mesh=_MESH,
        scratch_types=[
            pltpu.VMEM((EPT,), jnp.int32),
            pltpu.VMEM((NCHUNK, 128), jnp.int32),
            pltpu.VMEM((128, CIN), jnp.float32),
            pltpu.VMEM((128, CIN), jnp.float32),
            pltpu.VMEM((128, CIN), jnp.float32),
            pltpu.VMEM_SHARED((S1_ROWS, CIN), jnp.float32),
            pltpu.SemaphoreType.DMA,
            pltpu.SemaphoreType.DMA,
        ],
    )(x, sd_p, z2d)


def _sc_conv2_body(h2_hbm, sd_hbm, z64_hbm, z1d_hbm,
                   s2_out, c3_out,
                   srcv, d2b, c3b, rows0, rows1, onesv, zb, zb1,
                   s2acc, c3acc, sem0, sem1):
    cid = lax.axis_index("c")
    sid = lax.axis_index("s")
    wid = sid * _i32(2) + cid
    base = wid * _i32(EPT)

    pltpu.sync_copy(sd_hbm.at[pl.ds(base, EPT)], srcv)
    pltpu.sync_copy(z64_hbm, zb)
    pltpu.sync_copy(z1d_hbm, zb1)
    for k in range(8):
        onesv[pl.ds(k * 16, 16)] = jnp.ones((16,), jnp.float32)

    r0 = sid * _i32(S2_ROWS // 16)
    pltpu.sync_copy(zb.at[pl.ds(0, S2_ROWS // 16)],
                    s2acc.at[pl.ds(r0, S2_ROWS // 16)])
    f0 = sid * _i32(C3_STRIPE)
    pltpu.sync_copy(zb1, c3acc.at[pl.ds(f0, C3_STRIPE)])

    def idx_body(j, _):
        off = pl.multiple_of(j * _i32(128), 128)
        for k in range(8):
            pck = srcv[pl.ds(off + _i32(k * 16), 16)]
            sv = pck & _i32(0x3FFF)
            d = lax.shift_right_logical(pck, _i32(14))
            srcv[pl.ds(off + _i32(k * 16), 16)] = lax.shift_right_logical(
                sv, _i32(2))
            d2b[j, pl.ds(k * 16, 16)] = lax.shift_right_logical(d, _i32(3))
            s3 = lax.shift_right_logical(sv, _i32(4))
            c3 = lax.shift_right_logical(d, _i32(5)) * _i32(625) + s3
            c3b[j, pl.ds(k * 16, 16)] = jnp.where(
                d < _i32(N0), c3, _i32(196000) + s3)
        return _i32(0)

    lax.fori_loop(_i32(0), _i32(NCHUNK), idx_body, _i32(0))
    plsc.subcore_barrier()

    def gather(j, buf, sem):
        off = pl.multiple_of(j * _i32(128), 128)
        pltpu.async_copy(h2_hbm.at[srcv.at[pl.ds(off, 128)]], buf, sem)

    def wait(buf, sem):
        pltpu.make_async_copy(h2_hbm.at[pl.ds(0, 128)], buf, sem).wait()

    def scatter(j, buf):
        pltpu.sync_copy(buf, s2acc.at[d2b.at[j]], add=True)
        pltpu.sync_copy(onesv, c3acc.at[c3b.at[j]], add=True)

    gather(_i32(0), rows0, sem0)

    def main_body(i, _):
        j0 = i * _i32(2)
        gather(j0 + _i32(1), rows1, sem1)
        wait(rows0, sem0)
        scatter(j0, rows0)

        @pl.when(i < _i32(NCHUNK // 2 - 1))
        def _():
            gather(j0 + _i32(2), rows0, sem0)

        wait(rows1, sem1)
        scatter(j0 + _i32(1), rows1)
        return _i32(0)

    lax.fori_loop(_i32(0), _i32(NCHUNK // 2), main_body, _i32(0))
    plsc.subcore_barrier()

    pltpu.sync_copy(s2acc.at[pl.ds(r0, S2_ROWS // 16)],
                    s2_out.at[cid, pl.ds(r0, S2_ROWS // 16)])
    pltpu.sync_copy(c3acc.at[pl.ds(f0, C3_STRIPE)],
                    c3_out.at[cid, pl.ds(f0, C3_STRIPE)])


@jax.jit
def _sc_conv2(h2t, sd_p):
    z64 = jnp.zeros((128, 64), jnp.float32)
    z1d = jnp.zeros((C3_STRIPE,), jnp.float32)
    return pl.kernel(
        _sc_conv2_body,
        out_type=(
            jax.ShapeDtypeStruct((2, S2_ROWS, 64), jnp.float32),
            jax.ShapeDtypeStruct((2, C3_FLAT), jnp.float32),
        ),
        mesh=_MESH,
        scratch_types=[
            pltpu.VMEM((EPT,), jnp.int32),
            pltpu.VMEM((NCHUNK, 128), jnp.int32),
            pltpu.VMEM((NCHUNK, 128), jnp.int32),
            pltpu.VMEM((128, 64), jnp.float32),
            pltpu.VMEM((128, 64), jnp.float32),
            pltpu.VMEM((128,), jnp.float32),
            pltpu.VMEM((128, 64), jnp.float32),
            pltpu.VMEM((C3_STRIPE,), jnp.float32),
            pltpu.VMEM_SHARED((S2_ROWS, 64), jnp.float32),
            pltpu.VMEM_SHARED((C3_FLAT,), jnp.float32),
            pltpu.SemaphoreType.DMA,
            pltpu.SemaphoreType.DMA,
        ],
        compiler_params=pltpu.CompilerParams(use_tc_tiling_on_sc=False),
    )(h2t, sd_p, z64, z1d)


def _bn_relu(h, g, b):
    mu = jnp.mean(h, axis=0, keepdims=True)
    var = jnp.mean((h - mu) ** 2, axis=0, keepdims=True)
    return jnp.maximum((h - mu) * lax.rsqrt(var + 1e-5) * g + b, 0.0)


def _tc_b_body(s1_ref, xp_ref, w1_ref, w1s_ref, b1_ref, g1_ref, be1_ref,
               out_ref):
    s1 = s1_ref[0, :N1, :] + s1_ref[1, :N1, :]
    xp = xp_ref[...]
    px = xp[:, :CIN] + xp[:, CIN:]
    h = (jnp.dot(s1, w1_ref[...], preferred_element_type=jnp.float32,
             precision=lax.Precision.HIGHEST)
         + jnp.dot(px, w1s_ref[...], preferred_element_type=jnp.float32,
             precision=lax.Precision.HIGHEST)
         + b1_ref[...])
    out_ref[...] = _bn_relu(h, g1_ref[...], be1_ref[...])


@jax.jit
def _tc_b(s1p, xp, w1, w1s, b1, g1, be1):
    return pl.pallas_call(
        _tc_b_body,
        out_shape=jax.ShapeDtypeStruct((N1, 64), jnp.float32),
    )(s1p, xp, w1, w1s, b1, g1, be1)


def _tc_pool_body(h1r_ref, out_ref):
    h1r = h1r_ref[...]
    h2 = (h1r[:, :64] + h1r[:, 64:]) * 0.5
    out_ref[:N2, :] = h2
    out_ref[N2:, :] = jnp.zeros((H2_ROWS - N2, 64), jnp.float32)


@jax.jit
def _tc_pool(h1r):
    return pl.pallas_call(
        _tc_pool_body,
        out_shape=jax.ShapeDtypeStruct((H2_ROWS, 64), jnp.float32),
    )(h1r)


def _tc_d1_body(s2_ref, h1q_ref, w2_ref, w2s_ref, b2_ref, g2_ref, be2_ref,
                out_ref):
    f32 = jnp.float32
    s2 = s2_ref[0, :N3, :] + s2_ref[1, :N3, :]
    h1q = h1q_ref[...]
    p2 = (h1q[:, :64] + h1q[:, 64:128] + h1q[:, 128:192]
          + h1q[:, 192:]) * 0.5
    h = (jnp.dot(s2, w2_ref[...], preferred_element_type=f32,
             precision=lax.Precision.HIGHEST)
         + jnp.dot(p2, w2s_ref[...], preferred_element_type=f32,
             precision=lax.Precision.HIGHEST)
         + b2_ref[...])
    out_ref[...] = _bn_relu(h, g2_ref[...], be2_ref[...])


@jax.jit
def _tc_d1(s2p, h1q, w2, w2s, b2, g2, be2):
    return pl.pallas_call(
        _tc_d1_body,
        out_shape=jax.ShapeDtypeStruct((N3, 128), jnp.float32),
    )(s2p, h1q, w2, w2s, b2, g2, be2)


def _tc_d2_body(h3r_ref, h3q_ref, c3_ref, w3_ref, w3s_ref, b3_ref, g3_ref,
                be3_ref, out_ref):
    f32 = jnp.float32
    h3r = h3r_ref[...]
    h4 = (h3r[:, :128] + h3r[:, 128:]) * 0.5
    c3 = c3_ref[0] + c3_ref[1]
    m = jnp.dot(c3, h4, preferred_element_type=f32,
             precision=lax.Precision.HIGHEST)
    h3q = h3q_ref[...]
    p4 = (h3q[:, :128] + h3q[:, 128:256] + h3q[:, 256:384]
          + h3q[:, 384:]) * 0.5
    h = (jnp.dot(m, w3_ref[...], preferred_element_type=f32,
             precision=lax.Precision.HIGHEST)
         + jnp.dot(p4, w3s_ref[...], preferred_element_type=f32,
             precision=lax.Precision.HIGHEST)
         + b3_ref[...])
    out_ref[...] = _bn_relu(h, g3_ref[...], be3_ref[...])


@jax.jit
def _tc_d2(h3r, h3q, c3p, w3, w3s, b3, g3, be3):
    return pl.pallas_call(
        _tc_d2_body,
        out_shape=jax.ShapeDtypeStruct((N5, 256), jnp.float32),
    )(h3r, h3q, c3p, w3, w3s, b3, g3, be3)


def _tc_d3_body(gq_ref, b6_ref, wl_ref, bl_ref, out_ref):
    f32 = jnp.float32
    gq = gq_ref[...]
    s6 = gq[:, :256] + gq[:, 256:]
    ridx = lax.broadcasted_iota(jnp.int32, (N6, 1), 0)
    h6 = s6 * jnp.where(ridx < _i32(N6 - 1), f32(0.5), f32(1.0))
    b6 = b6_ref[...]
    parts = []
    for b in range(8):
        mb = jnp.where(b6 == _i32(b), h6, -jnp.inf)
        parts.append(jnp.max(mb, axis=0, keepdims=True))
    pooled = jnp.concatenate(parts, axis=0)
    pooled = jnp.where(jnp.isfinite(pooled), pooled, f32(0.0))
    out_ref[...] = (jnp.dot(pooled, wl_ref[...], preferred_element_type=f32,
             precision=lax.Precision.HIGHEST)
                    + bl_ref[...])


@jax.jit
def _tc_d3(gq, b6, wl, bl):
    return pl.pallas_call(
        _tc_d3_body,
        out_shape=jax.ShapeDtypeStruct((8, 40), jnp.float32),
    )(gq, b6, wl, bl)


def kernel(x, edge_index, batch_ids, W1, W1s, b1, g1, be1, W2, W2s, b2, g2,
           be2, W3, W3s, b3, g3, be3, Wl, bl):
    x = x.astype(jnp.float32)
    src = edge_index[0].astype(jnp.int32).reshape(NW, NE // NW)
    dst = edge_index[1].astype(jnp.int32).reshape(NW, NE // NW)

    k = jnp.arange(PAD_PER_TILE, dtype=jnp.int32)
    pad_src = jnp.broadcast_to(k % N0, (NW, PAD_PER_TILE))
    pad_dst = jnp.broadcast_to(N0 + (k % 16), (NW, PAD_PER_TILE))
    src_p = jnp.concatenate([src, pad_src], axis=1).reshape(NEP)
    dst_p = jnp.concatenate([dst, pad_dst], axis=1).reshape(NEP)
    sd_p = src_p | (dst_p << 14)

    s1p = _sc_conv1(x, sd_p)

    xp = x.reshape(N1, 2 * CIN)
    h1 = _tc_b(s1p, xp, W1, W1s, b1.reshape(1, 64), g1.reshape(1, 64),
               be1.reshape(1, 64))

    h1r = h1.reshape(N2, 128)
    h1q = h1.reshape(N3, 256)

    h2t = _tc_pool(h1r)
    s2p, c3p = _sc_conv2(h2t, sd_p)
    c3m = c3p[:, :N5 * N4].reshape(2, N5, N4)

    h3 = _tc_d1(s2p, h1q, W2, W2s, b2.reshape(1, 128), g2.reshape(1, 128),
                be2.reshape(1, 128))
    h3r = h3.reshape(N4, 256)
    h3q = jnp.concatenate(
        [h3, jnp.zeros((2, 128), jnp.float32)], axis=0).reshape(N5, 512)

    gg = _tc_d2(h3r, h3q, c3m, W3, W3s, b3.reshape(1, 256),
                g3.reshape(1, 256), be3.reshape(1, 256))
    gq = jnp.concatenate(
        [gg, jnp.zeros((1, 256), jnp.float32)], axis=0).reshape(N6, 512)

    b6 = batch_ids[::64][:N6].astype(jnp.int32).reshape(N6, 1)
    return _tc_d3(gq, b6, Wl, bl.reshape(1, 40))


# speedup vs baseline: 22.1875x; 1.0833x over previous
import jax
import jax.numpy as jnp
from jax import lax
from jax.experimental import pallas as pl
from jax.experimental.pallas import tpu as pltpu
from jax.experimental.pallas import tpu_sc as plsc

N0, N1, N2, N3, N4, N5, N6 = 10000, 5000, 2500, 1250, 625, 313, 157
CIN = 128
NE = 320000
NW = 32
EPT = 10240
NCHUNK = EPT // 128
NEP = EPT * NW
PAD_PER_TILE = EPT - NE // NW

S1_ROWS = 5120
S1_STRIPE = S1_ROWS // 16
C3_FLAT = 204800
C3_STRIPE = C3_FLAT // 16
S2_ROWS = 1280
H2_ROWS = 2560

_i32 = jnp.int32

_MESH = plsc.VectorSubcoreMesh(core_axis_name="c", subcore_axis_name="s")


def _sc_conv1_body(x_hbm, sd_hbm, z2d_hbm,
                   s1_out,
                   srcv, d1b, rows0, rows1, zb,
                   s1acc, sem0, sem1):
    cid = lax.axis_index("c")
    sid = lax.axis_index("s")
    wid = sid * _i32(2) + cid
    base = wid * _i32(EPT)

    pltpu.sync_copy(sd_hbm.at[pl.ds(base, EPT)], srcv)
    pltpu.sync_copy(z2d_hbm, zb)

    r0 = sid * _i32(S1_STRIPE)
    pltpu.sync_copy(zb, s1acc.at[pl.ds(r0, 128)])
    pltpu.sync_copy(zb, s1acc.at[pl.ds(r0 + _i32(128), 128)])
    pltpu.sync_copy(zb.at[pl.ds(0, S1_STRIPE - 256)],
                    s1acc.at[pl.ds(r0 + _i32(256), S1_STRIPE - 256)])

    def idx_body(j, _):
        off = pl.multiple_of(j * _i32(128), 128)
        for k in range(8):
            pck = srcv[pl.ds(off + _i32(k * 16), 16)]
            d = lax.shift_right_logical(pck, _i32(14))
            d1b[j, pl.ds(k * 16, 16)] = lax.shift_right_logical(d, _i32(1))
            srcv[pl.ds(off + _i32(k * 16), 16)] = pck & _i32(0x3FFF)
        return _i32(0)

    lax.fori_loop(_i32(0), _i32(NCHUNK), idx_body, _i32(0))
    plsc.subcore_barrier()

    def gather(j, buf, sem):
        off = pl.multiple_of(j * _i32(128), 128)
        pltpu.async_copy(x_hbm.at[srcv.at[pl.ds(off, 128)]], buf, sem)

    def wait(buf, sem):
        pltpu.make_async_copy(x_hbm.at[pl.ds(0, 128)], buf, sem).wait()

    def scatter(j, buf):
        pltpu.sync_copy(buf, s1acc.at[d1b.at[j]], add=True)

    gather(_i32(0), rows0, sem0)

    def main_body(i, _):
        j0 = i * _i32(2)
        gather(j0 + _i32(1), rows1, sem1)
        wait(rows0, sem0)
        scatter(j0, rows0)

        @pl.when(i < _i32(NCHUNK // 2 - 1))
        def _():
            gather(j0 + _i32(2), rows0, sem0)

        wait(rows1, sem1)
        scatter(j0 + _i32(1), rows1)
        return _i32(0)

    lax.fori_loop(_i32(0), _i32(NCHUNK // 2), main_body, _i32(0))
    plsc.subcore_barrier()

    pltpu.sync_copy(s1acc.at[pl.ds(r0, S1_STRIPE)],
                    s1_out.at[cid, pl.ds(r0, S1_STRIPE)])


@jax.jit
def _sc_conv1(x, sd_p):
    z2d = jnp.zeros((128, CIN), jnp.float32)
    return pl.kernel(
        _sc_conv1_body,
        out_type=jax.ShapeDtypeStruct((2, S1_ROWS, CIN), jnp.float32),
        mesh=_MESH,
        scratch_types=[
            pltpu.VMEM((EPT,), jnp.int32),
            pltpu.VMEM((NCHUNK, 128), jnp.int32),
            pltpu.VMEM((128, CIN), jnp.float32),
            pltpu.VMEM((128, CIN), jnp.float32),
            pltpu.VMEM((128, CIN), jnp.float32),
            pltpu.VMEM_SHARED((S1_ROWS, CIN), jnp.float32),
            pltpu.SemaphoreType.DMA,
            pltpu.SemaphoreType.DMA,
        ],
    )(x, sd_p, z2d)


def _sc_conv2_body(h2_hbm, sd_hbm, z64_hbm, z1d_hbm,
                   s2_out, c3_out,
                   srcv, d2b, c3b, rows0, rows1, onesv, zb, zb1,
                   s2acc, c3acc, sem0, sem1):
    cid = lax.axis_index("c")
    sid = lax.axis_index("s")
    wid = sid * _i32(2) + cid
    base = wid * _i32(EPT)

    pltpu.sync_copy(sd_hbm.at[pl.ds(base, EPT)], srcv)
    pltpu.sync_copy(z64_hbm, zb)
    pltpu.sync_copy(z1d_hbm, zb1)
    for k in range(8):
        onesv[pl.ds(k * 16, 16)] = jnp.ones((16,), jnp.float32)

    r0 = sid * _i32(S2_ROWS // 16)
    pltpu.sync_copy(zb.at[pl.ds(0, S2_ROWS // 16)],
                    s2acc.at[pl.ds(r0, S2_ROWS // 16)])
    f0 = sid * _i32(C3_STRIPE)
    pltpu.sync_copy(zb1, c3acc.at[pl.ds(f0, C3_STRIPE)])

    def idx_body(j, _):
        off = pl.multiple_of(j * _i32(128), 128)
        for k in range(8):
            pck = srcv[pl.ds(off + _i32(k * 16), 16)]
            sv = pck & _i32(0x3FFF)
            d = lax.shift_right_logical(pck, _i32(14))
            srcv[pl.ds(off + _i32(k * 16), 16)] = lax.shift_right_logical(
                sv, _i32(2))
            d2b[j, pl.ds(k * 16, 16)] = lax.shift_right_logical(d, _i32(3))
            s3 = lax.shift_right_logical(sv, _i32(4))
            c3 = lax.shift_right_logical(d, _i32(5)) * _i32(625) + s3
            c3b[j, pl.ds(k * 16, 16)] = jnp.where(
                d < _i32(N0), c3, _i32(196000) + s3)
        return _i32(0)

    lax.fori_loop(_i32(0), _i32(NCHUNK), idx_body, _i32(0))
    plsc.subcore_barrier()

    def gather(j, buf, sem):
        off = pl.multiple_of(j * _i32(128), 128)
        pltpu.async_copy(h2_hbm.at[srcv.at[pl.ds(off, 128)]], buf, sem)

    def wait(buf, sem):
        pltpu.make_async_copy(h2_hbm.at[pl.ds(0, 128)], buf, sem).wait()

    def scatter(j, buf):
        pltpu.sync_copy(buf, s2acc.at[d2b.at[j]], add=True)
        pltpu.sync_copy(onesv, c3acc.at[c3b.at[j]], add=True)

    gather(_i32(0), rows0, sem0)

    def main_body(i, _):
        j0 = i * _i32(2)
        gather(j0 + _i32(1), rows1, sem1)
        wait(rows0, sem0)
        scatter(j0, rows0)

        @pl.when(i < _i32(NCHUNK // 2 - 1))
        def _():
            gather(j0 + _i32(2), rows0, sem0)

        wait(rows1, sem1)
        scatter(j0 + _i32(1), rows1)
        return _i32(0)

    lax.fori_loop(_i32(0), _i32(NCHUNK // 2), main_body, _i32(0))
    plsc.subcore_barrier()

    pltpu.sync_copy(s2acc.at[pl.ds(r0, S2_ROWS // 16)],
                    s2_out.at[cid, pl.ds(r0, S2_ROWS // 16)])
    pltpu.sync_copy(c3acc.at[pl.ds(f0, C3_STRIPE)],
                    c3_out.at[cid, pl.ds(f0, C3_STRIPE)])


@jax.jit
def _sc_conv2(h2t, sd_p):
    z64 = jnp.zeros((128, 64), jnp.float32)
    z1d = jnp.zeros((C3_STRIPE,), jnp.float32)
    return pl.kernel(
        _sc_conv2_body,
        out_type=(
            jax.ShapeDtypeStruct((2, S2_ROWS, 64), jnp.float32),
            jax.ShapeDtypeStruct((2, C3_FLAT), jnp.float32),
        ),
        mesh=_MESH,
        scratch_types=[
            pltpu.VMEM((EPT,), jnp.int32),
            pltpu.VMEM((NCHUNK, 128), jnp.int32),
            pltpu.VMEM((NCHUNK, 128), jnp.int32),
            pltpu.VMEM((128, 64), jnp.float32),
            pltpu.VMEM((128, 64), jnp.float32),
            pltpu.VMEM((128,), jnp.float32),
            pltpu.VMEM((128, 64), jnp.float32),
            pltpu.VMEM((C3_STRIPE,), jnp.float32),
            pltpu.VMEM_SHARED((S2_ROWS, 64), jnp.float32),
            pltpu.VMEM_SHARED((C3_FLAT,), jnp.float32),
            pltpu.SemaphoreType.DMA,
            pltpu.SemaphoreType.DMA,
        ],
        compiler_params=pltpu.CompilerParams(use_tc_tiling_on_sc=False),
    )(h2t, sd_p, z64, z1d)


def _bn_relu(h, g, b):
    mu = jnp.mean(h, axis=0, keepdims=True)
    var = jnp.mean((h - mu) ** 2, axis=0, keepdims=True)
    return jnp.maximum((h - mu) * lax.rsqrt(var + 1e-5) * g + b, 0.0)


def _tc_b_body(s1_ref, xp_ref, w1_ref, w1s_ref, b1_ref, g1_ref, be1_ref,
               out_ref):
    s1 = s1_ref[0, :N1, :] + s1_ref[1, :N1, :]
    xp = xp_ref[...]
    px = xp[:, :CIN] + xp[:, CIN:]
    h = (jnp.dot(s1, w1_ref[...], preferred_element_type=jnp.float32,
             precision=lax.Precision.HIGHEST)
         + jnp.dot(px, w1s_ref[...], preferred_element_type=jnp.float32,
             precision=lax.Precision.HIGHEST)
         + b1_ref[...])
    out_ref[...] = _bn_relu(h, g1_ref[...], be1_ref[...])


@jax.jit
def _tc_b(s1p, xp, w1, w1s, b1, g1, be1):
    return pl.pallas_call(
        _tc_b_body,
        out_shape=jax.ShapeDtypeStruct((N1, 64), jnp.float32),
    )(s1p, xp, w1, w1s, b1, g1, be1)


def _tc_pool_body(h1r_ref, out_ref):
    h1r = h1r_ref[...]
    h2 = (h1r[:, :64] + h1r[:, 64:]) * 0.5
    out_ref[:N2, :] = h2
    out_ref[N2:, :] = jnp.zeros((H2_ROWS - N2, 64), jnp.float32)


@jax.jit
def _tc_pool(h1r):
    return pl.pallas_call(
        _tc_pool_body,
        out_shape=jax.ShapeDtypeStruct((H2_ROWS, 64), jnp.float32),
    )(h1r)


def _tc_d1_body(s2_ref, h1q_ref, w2_ref, w2s_ref, b2_ref, g2_ref, be2_ref,
                out_ref):
    f32 = jnp.float32
    s2 = s2_ref[0, :N3, :] + s2_ref[1, :N3, :]
    h1q = h1q_ref[...]
    p2 = (h1q[:, :64] + h1q[:, 64:128] + h1q[:, 128:192]
          + h1q[:, 192:]) * 0.5
    h = (jnp.dot(s2, w2_ref[...], preferred_element_type=f32,
             precision=lax.Precision.HIGHEST)
         + jnp.dot(p2, w2s_ref[...], preferred_element_type=f32,
             precision=lax.Precision.HIGHEST)
         + b2_ref[...])
    out_ref[...] = _bn_relu(h, g2_ref[...], be2_ref[...])


@jax.jit
def _tc_d1(s2p, h1q, w2, w2s, b2, g2, be2):
    return pl.pallas_call(
        _tc_d1_body,
        out_shape=jax.ShapeDtypeStruct((N3, 128), jnp.float32),
    )(s2p, h1q, w2, w2s, b2, g2, be2)


def _tc_d2_body(h3r_ref, h3q_ref, c3_ref, w3_ref, w3s_ref, b3_ref, g3_ref,
                be3_ref, out_ref):
    f32 = jnp.float32
    h3r = h3r_ref[...]
    h4 = (h3r[:, :128] + h3r[:, 128:]) * 0.5
    c3 = c3_ref[0] + c3_ref[1]
    m = jnp.dot(c3, h4, preferred_element_type=f32,
             precision=lax.Precision.HIGHEST)
    h3q = h3q_ref[...]
    p4 = (h3q[:, :128] + h3q[:, 128:256] + h3q[:, 256:384]
          + h3q[:, 384:]) * 0.5
    h = (jnp.dot(m, w3_ref[...], preferred_element_type=f32,
             precision=lax.Precision.HIGHEST)
         + jnp.dot(p4, w3s_ref[...], preferred_element_type=f32,
             precision=lax.Precision.HIGHEST)
         + b3_ref[...])
    out_ref[...] = _bn_relu(h, g3_ref[...], be3_ref[...])


@jax.jit
def _tc_d2(h3r, h3q, c3p, w3, w3s, b3, g3, be3):
    return pl.pallas_call(
        _tc_d2_body,
        out_shape=jax.ShapeDtypeStruct((N5, 256), jnp.float32),
    )(h3r, h3q, c3p, w3, w3s, b3, g3, be3)


def _tc_d3_body(gq_ref, b6_ref, wl_ref, bl_ref, out_ref):
    f32 = jnp.float32
    gq = gq_ref[...]
    s6 = gq[:, :256] + gq[:, 256:]
    ridx = lax.broadcasted_iota(jnp.int32, (N6, 1), 0)
    h6 = s6 * jnp.where(ridx < _i32(N6 - 1), f32(0.5), f32(1.0))
    b6 = b6_ref[...]
    parts = []
    for b in range(8):
        mb = jnp.where(b6 == _i32(b), h6, -jnp.inf)
        parts.append(jnp.max(mb, axis=0, keepdims=True))
    pooled = jnp.concatenate(parts, axis=0)
    pooled = jnp.where(jnp.isfinite(pooled), pooled, f32(0.0))
    out_ref[...] = (jnp.dot(pooled, wl_ref[...], preferred_element_type=f32,
             precision=lax.Precision.HIGHEST)
                    + bl_ref[...])


@jax.jit
def _tc_d3(gq, b6, wl, bl):
    return pl.pallas_call(
        _tc_d3_body,
        out_shape=jax.ShapeDtypeStruct((8, 40), jnp.float32),
    )(gq, b6, wl, bl)


def kernel(x, edge_index, batch_ids, W1, W1s, b1, g1, be1, W2, W2s, b2, g2,
           be2, W3, W3s, b3, g3, be3, Wl, bl):
    x = x.astype(jnp.float32)
    src = edge_index[0].astype(jnp.int32)
    dst = edge_index[1].astype(jnp.int32)

    k = jnp.arange(NEP - NE, dtype=jnp.int32)
    pad_sd = (k % N0) | ((N0 + (k % 16)) << 14)
    sd_p = jnp.concatenate([src | (dst << 14), pad_sd])

    s1p = _sc_conv1(x, sd_p)

    xp = x.reshape(N1, 2 * CIN)
    h1 = _tc_b(s1p, xp, W1, W1s, b1.reshape(1, 64), g1.reshape(1, 64),
               be1.reshape(1, 64))

    h1r = h1.reshape(N2, 128)
    h1q = h1.reshape(N3, 256)

    h2t = _tc_pool(h1r)
    s2p, c3p = _sc_conv2(h2t, sd_p)
    c3m = c3p[:, :N5 * N4].reshape(2, N5, N4)

    h3 = _tc_d1(s2p, h1q, W2, W2s, b2.reshape(1, 128), g2.reshape(1, 128),
                be2.reshape(1, 128))
    h3r = h3.reshape(N4, 256)
    h3q = jnp.concatenate(
        [h3, jnp.zeros((2, 128), jnp.float32)], axis=0).reshape(N5, 512)

    gg = _tc_d2(h3r, h3q, c3m, W3, W3s, b3.reshape(1, 256),
                g3.reshape(1, 256), be3.reshape(1, 256))
    gq = jnp.concatenate(
        [gg, jnp.zeros((1, 256), jnp.float32)], axis=0).reshape(N6, 512)

    b6 = batch_ids[::64][:N6].astype(jnp.int32).reshape(N6, 1)
    return _tc_d3(gq, b6, Wl, bl.reshape(1, 40))
